# baseline (speedup 1.0000x reference)
import jax
import jax.numpy as jnp
from jax.experimental import pallas as pl
from jax.experimental.pallas import tpu as pltpu

WINDOW = 512
HEAD_DIM = 64
NUM_HEADS = 12
BQ = 256
KB = WINDOW + BQ


def _attn_kernel(x_ref, o_ref, kc_ref, vc_ref, q_raw, k_raw, v_raw,
                 q_sem, k_sem, v_sem):
    ib = pl.program_id(0)
    i = pl.program_id(1)
    nq = pl.num_programs(1)
    scale = 1.0 / (HEAD_DIM ** 0.5)
    kstart = jnp.maximum(i - 2, 0) * BQ
    slot = jax.lax.rem(i, 2)
    nslot = jax.lax.rem(i + 1, 2)

    def chunk_copies(j, s):
        rows = pl.ds(j * BQ, BQ)
        return (
            pltpu.make_async_copy(x_ref.at[ib, rows, 0], q_raw.at[s], q_sem.at[s]),
            pltpu.make_async_copy(x_ref.at[ib, rows, 1], k_raw.at[s], k_sem.at[s]),
            pltpu.make_async_copy(x_ref.at[ib, rows, 2], v_raw.at[s], v_sem.at[s]),
        )

    @pl.when(i == 0)
    def _start_first():
        for cp in chunk_copies(0, slot):
            cp.start()
        zeros = jnp.zeros((KB - BQ, NUM_HEADS * HEAD_DIM), jnp.float32)
        kc_ref[pl.ds(BQ, KB - BQ), :] = zeros
        vc_ref[pl.ds(BQ, KB - BQ), :] = zeros

    for cp in chunk_copies(i, slot):
        cp.wait()

    @pl.when(i + 1 < nq)
    def _prefetch_next():
        for cp in chunk_copies(i + 1, nslot):
            cp.start()

    qt = jnp.transpose(q_raw[slot], (1, 0, 2))
    kt = jnp.transpose(k_raw[slot], (1, 0, 2))
    vt = jnp.transpose(v_raw[slot], (1, 0, 2))
    rows_i = pl.ds(i * BQ, BQ)
    for hh in range(NUM_HEADS):
        c0 = hh * HEAD_DIM
        kc_ref[rows_i, c0:c0 + HEAD_DIM] = kt[hh]
        vc_ref[rows_i, c0:c0 + HEAD_DIM] = vt[hh]

    q_idx = i * BQ + jax.lax.broadcasted_iota(jnp.int32, (BQ, KB), 0)
    kv_idx = kstart + jax.lax.broadcasted_iota(jnp.int32, (BQ, KB), 1)
    diff = q_idx - kv_idx
    mask = (diff >= 0) & (diff <= WINDOW)
    bias = jnp.where(mask, jnp.float32(0), jnp.float32(float("-inf")))
    outs = []
    for hh in range(NUM_HEADS):
        c0 = hh * HEAD_DIM
        qh = qt[hh] * scale
        kh = kc_ref[pl.ds(kstart, KB), c0:c0 + HEAD_DIM]
        vh = vc_ref[pl.ds(kstart, KB), c0:c0 + HEAD_DIM]
        s = jax.lax.dot_general(
            qh, kh, (((1,), (1,)), ((), ())),
            preferred_element_type=jnp.float32) + bias
        m = jnp.max(s, axis=-1, keepdims=True)
        p = jnp.exp(s - m)
        denom = jnp.sum(p, axis=-1, keepdims=True)
        oh = jax.lax.dot_general(
            p, vh, (((1,), (0,)), ((), ())),
            preferred_element_type=jnp.float32)
        outs.append(oh * (1.0 / denom))
    ot = jnp.stack(outs, axis=0)
    o_ref[0] = jnp.transpose(ot, (1, 0, 2))


def kernel(qkv):
    b, l, three, h, e = qkv.shape
    nq = l // BQ

    return pl.pallas_call(
        _attn_kernel,
        grid=(b, nq),
        in_specs=[pl.BlockSpec(memory_space=pltpu.MemorySpace.HBM)],
        out_specs=pl.BlockSpec((1, BQ, h, e), lambda ib, i: (ib, i, 0, 0)),
        out_shape=jax.ShapeDtypeStruct((b, l, h, e), jnp.float32),
        scratch_shapes=[
            pltpu.VMEM((l, h * e), jnp.float32),
            pltpu.VMEM((l, h * e), jnp.float32),
            pltpu.VMEM((2, BQ, h, e), jnp.float32),
            pltpu.VMEM((2, BQ, h, e), jnp.float32),
            pltpu.VMEM((2, BQ, h, e), jnp.float32),
            pltpu.SemaphoreType.DMA((2,)),
            pltpu.SemaphoreType.DMA((2,)),
            pltpu.SemaphoreType.DMA((2,)),
        ],
    )(qkv)


# speedup vs baseline: 1.6238x; 1.6238x over previous
import jax
import jax.numpy as jnp
from jax.experimental import pallas as pl
from jax.experimental.pallas import tpu as pltpu

WINDOW = 512
HEAD_DIM = 64
NUM_HEADS = 12
BQ = 256
KB = WINDOW + BQ


def _attn_kernel(q_ref, k_ref, v_ref, o_ref):
    i = pl.program_id(1)
    scale = 1.0 / (HEAD_DIM ** 0.5)
    kstart = jnp.maximum(i - 2, 0) * BQ
    q_idx = i * BQ + jax.lax.broadcasted_iota(jnp.int32, (BQ, KB), 0)
    kv_idx = kstart + jax.lax.broadcasted_iota(jnp.int32, (BQ, KB), 1)
    diff = q_idx - kv_idx
    mask = (diff >= 0) & (diff <= WINDOW)
    bias = jnp.where(mask, jnp.float32(0), jnp.float32(float("-inf")))
    for hh in range(NUM_HEADS):
        c0 = hh * HEAD_DIM
        qh = q_ref[0, :, c0:c0 + HEAD_DIM] * scale
        kh = k_ref[0, pl.ds(kstart, KB), c0:c0 + HEAD_DIM]
        vh = v_ref[0, pl.ds(kstart, KB), c0:c0 + HEAD_DIM]
        s = jax.lax.dot_general(
            qh, kh, (((1,), (1,)), ((), ())),
            preferred_element_type=jnp.float32) + bias
        m = jnp.max(s, axis=-1, keepdims=True)
        p = jnp.exp(s - m)
        denom = jnp.sum(p, axis=-1, keepdims=True)
        oh = jax.lax.dot_general(
            p, vh, (((1,), (0,)), ((), ())),
            preferred_element_type=jnp.float32)
        o_ref[0, :, c0:c0 + HEAD_DIM] = oh * (1.0 / denom)


def kernel(qkv):
    b, l, three, h, e = qkv.shape
    ch = h * e
    x = qkv.reshape(b, l, three * ch)
    nq = l // BQ

    out = pl.pallas_call(
        _attn_kernel,
        grid=(b, nq),
        in_specs=[
            pl.BlockSpec((1, BQ, ch), lambda ib, i: (ib, i, 0)),
            pl.BlockSpec((1, l, ch), lambda ib, i: (ib, 0, 1)),
            pl.BlockSpec((1, l, ch), lambda ib, i: (ib, 0, 2)),
        ],
        out_specs=pl.BlockSpec((1, BQ, ch), lambda ib, i: (ib, i, 0)),
        out_shape=jax.ShapeDtypeStruct((b, l, ch), jnp.float32),
    )(x, x, x)

    return out.reshape(b, l, h, e)
